# Initial kernel scaffold; baseline (speedup 1.0000x reference)
import jax
import jax.numpy as jnp
from jax.experimental import pallas as pl


def kernel(p, params):
    raise NotImplementedError("write your pallas kernel here")


# speedup vs baseline: 38.3978x; 38.3978x over previous
import jax
import jax.numpy as jnp
from jax.experimental import pallas as pl

K_NN = 16
STRIDE = 8


def _mm(a, b):
    return jax.lax.dot_general(a, b, (((1,), (0,)), ((), ())),
                               preferred_element_type=jnp.float32)


def _tr(a):
    n = a.shape[0]
    eye = (jax.lax.broadcasted_iota(jnp.int32, (n, n), 0) ==
           jax.lax.broadcasted_iota(jnp.int32, (n, n), 1)).astype(jnp.float32)
    return jax.lax.dot_general(a, eye, (((0,), (0,)), ((), ())),
                               preferred_element_type=jnp.float32)


def _d2(qpos, posT):
    acc = None
    for c in range(3):
        dc = qpos[:, c:c + 1] - posT[c:c + 1, :]
        acc = dc * dc if acc is None else acc + dc * dc
    return acc


def _knn_gather(d2, payload):
    outs = []
    for _ in range(K_NN):
        m = jnp.min(d2, axis=1, keepdims=True)
        oh = d2 == m
        outs.append(_mm(oh.astype(jnp.float32), payload))
        d2 = jnp.where(oh, jnp.float32(jnp.inf), d2)
    return outs


def _pt(pos, posT, x, W, qt):
    (Wq, Wk, Wv, Wp1, bp1, Wp2, bp2, Wg1, bg1, Wg2, bg2, Wo) = W
    n, d = x.shape
    q = _mm(x, Wq)
    k = _mm(x, Wk)
    v = _mm(x, Wv)
    kvp = jnp.concatenate([k, v, pos], axis=1)
    outs = []
    for qs in range(0, n, qt):
        pq = pos[qs:qs + qt]
        d2 = _d2(pq, posT)
        g16 = _knn_gather(d2, kvp)
        cat = jnp.concatenate(g16, axis=0)
        kj = cat[:, :d]
        vj = cat[:, d:2 * d]
        pj = cat[:, 2 * d:2 * d + 3]
        rel = jnp.concatenate([pq] * K_NN, axis=0) - pj
        pe = _mm(jax.nn.relu(_mm(rel, Wp1) + bp1), Wp2) + bp2
        qc = jnp.concatenate([q[qs:qs + qt]] * K_NN, axis=0)
        g = qc - kj + pe
        a = _mm(jax.nn.relu(_mm(g, Wg1) + bg1), Wg2) + bg2
        a3 = a.reshape(K_NN, qt, d)
        mx = jnp.max(a3, axis=0, keepdims=True)
        e = jnp.exp(a3 - mx)
        w = e / jnp.sum(e, axis=0, keepdims=True)
        sv = (vj + pe).reshape(K_NN, qt, d)
        o = jnp.sum(w * sv, axis=0)
        outs.append(_mm(o, Wo))
    out = outs[0] if len(outs) == 1 else jnp.concatenate(outs, axis=0)
    return x + out


def _td(pos, posT, x, W, b):
    n, d = x.shape
    m = n // STRIDE
    r = jax.lax.broadcasted_iota(jnp.int32, (m, n), 0)
    c = jax.lax.broadcasted_iota(jnp.int32, (m, n), 1)
    sel = (c == r * STRIDE).astype(jnp.float32)
    sub = _mm(sel, pos)
    subT = _tr(sub)
    d2 = _d2(sub, posT)
    g16 = _knn_gather(d2, x)
    cat = jnp.concatenate(g16, axis=0)
    h = jax.nn.relu(_mm(cat, W) + b)
    h3 = h.reshape(K_NN, m, h.shape[1])
    return sub, subT, jnp.max(h3, axis=0)


def _res(x, W1, b1, W2, b2):
    h = _mm(jax.nn.relu(_mm(x, W1) + b1), W2) + b2
    return jax.nn.relu(x + h)


def _body(p_ref, *refs):
    w_refs = refs[:-1]
    o_ref = refs[-1]
    ws = [r[...] for r in w_refs]
    it = iter(ws)

    def take(k):
        return tuple(next(it) for _ in range(k))

    Win, bin_ = take(2)
    t1 = take(12)
    Wl32, bl32, Wtd1, btd1, Wl64, bl64 = take(6)
    t2 = take(12)
    r11 = take(4)
    r12 = take(4)
    Wtd2, btd2 = take(2)
    r21 = take(4)
    r22 = take(4)
    Wf1, bf1, Wf2, bf2, Wf3, bf3 = take(6)

    posT = p_ref[0]
    pos = _tr(posT)
    x = jax.nn.relu(_mm(pos, Win) + bin_)
    x = _pt(pos, posT, x, t1, 256)
    x = _mm(x, Wl32) + bl32
    pos, posT, x = _td(pos, posT, x, Wtd1, btd1)
    x = _mm(x, Wl64) + bl64
    x = _pt(pos, posT, x, t2, 128)
    x = _res(x, *r11)
    x = _res(x, *r12)
    pos, posT, x = _td(pos, posT, x, Wtd2, btd2)
    x = _res(x, *r21)
    x = _res(x, *r22)
    g = jnp.max(x, axis=0, keepdims=True)
    h = jax.nn.relu(_mm(g, Wf1) + bf1)
    h = jax.nn.relu(_mm(h, Wf2) + bf2)
    o = _mm(h, Wf3) + bf3
    o = o - jnp.max(o, axis=1, keepdims=True)
    o = o - jnp.log(jnp.sum(jnp.exp(o), axis=1, keepdims=True))
    o_ref[0] = o


def _flat_weights(params):
    def b2(v):
        return v.reshape(1, -1)

    def ptw(t):
        return [t['Wq'], t['Wk'], t['Wv'], t['Wp1'], b2(t['bp1']),
                t['Wp2'], b2(t['bp2']), t['Wg1'], b2(t['bg1']),
                t['Wg2'], b2(t['bg2']), t['Wo']]

    def resw(r):
        return [r['W1'], b2(r['b1']), r['W2'], b2(r['b2'])]

    return ([params['Win'], b2(params['bin'])] + ptw(params['t1']) +
            [params['Wl32'], b2(params['bl32']),
             params['Wtd1'], b2(params['btd1']),
             params['Wl64'], b2(params['bl64'])] + ptw(params['t2']) +
            resw(params['r11']) + resw(params['r12']) +
            [params['Wtd2'], b2(params['btd2'])] +
            resw(params['r21']) + resw(params['r22']) +
            [params['Wf1'], b2(params['bf1']), params['Wf2'], b2(params['bf2']),
             params['Wf3'], b2(params['bf3'])])


def kernel(p, params):
    ws = _flat_weights(params)
    B = p.shape[0]
    out = pl.pallas_call(
        _body,
        grid=(B,),
        in_specs=[pl.BlockSpec((1, 3, 1024), lambda b: (b, 0, 0))] +
                 [pl.BlockSpec(w.shape, lambda b, nd=w.ndim: (0,) * nd)
                  for w in ws],
        out_specs=pl.BlockSpec((1, 1, 40), lambda b: (b, 0, 0)),
        out_shape=jax.ShapeDtypeStruct((B, 1, 40), jnp.float32),
    )(p, *ws)
    return out.reshape(B, 40)
